# baseline (speedup 1.0000x reference)
import functools

import jax
import jax.numpy as jnp
from jax.experimental import pallas as pl

N = 10000


def _gcn_body(block_m, emit_bf16_a, a_ref, hfull_ref, w_ref, b_ref, deg_ref,
              *out_refs):
    i = pl.program_id(0)
    if emit_bf16_a:
        out_ref, abf_ref = out_refs
        a16 = a_ref[...].astype(jnp.bfloat16)
        abf_ref[...] = a16
    else:
        (out_ref,) = out_refs
        a16 = a_ref[...]
        if a16.dtype != jnp.bfloat16:
            a16 = a16.astype(jnp.bfloat16)
    h16 = hfull_ref[...]
    if h16.dtype != jnp.bfloat16:
        h16 = h16.astype(jnp.bfloat16)
    hself = hfull_ref[pl.ds(i * block_m, block_m), :]
    pool = jax.lax.dot(a16, h16, preferred_element_type=jnp.float32)
    pool = pool + hself.astype(jnp.float32)
    lin = jax.lax.dot(pool.astype(jnp.bfloat16),
                      w_ref[...].astype(jnp.bfloat16),
                      preferred_element_type=jnp.float32)
    lin = lin + b_ref[...]
    out = jnp.maximum(lin / deg_ref[...], 0.0)
    out_ref[...] = out.astype(out_ref.dtype)


def _layer(a, hfull, deg, w, b, *, block_m, out_dtype, emit_bf16_a,
           interpret=False):
    fin = hfull.shape[1]
    fout = w.shape[1]
    grid = (N // block_m,)
    in_specs = [
        pl.BlockSpec((block_m, N), lambda i: (i, 0)),
        pl.BlockSpec((N, fin), lambda i: (0, 0)),
        pl.BlockSpec((fin, fout), lambda i: (0, 0)),
        pl.BlockSpec((1, fout), lambda i: (0, 0)),
        pl.BlockSpec((block_m, 1), lambda i: (i, 0)),
    ]
    if emit_bf16_a:
        out_shape = (
            jax.ShapeDtypeStruct((N, fout), out_dtype),
            jax.ShapeDtypeStruct((N, N), jnp.bfloat16),
        )
        out_specs = (
            pl.BlockSpec((block_m, fout), lambda i: (i, 0)),
            pl.BlockSpec((block_m, N), lambda i: (i, 0)),
        )
    else:
        out_shape = jax.ShapeDtypeStruct((N, fout), out_dtype)
        out_specs = pl.BlockSpec((block_m, fout), lambda i: (i, 0))
    return pl.pallas_call(
        functools.partial(_gcn_body, block_m, emit_bf16_a),
        grid=grid,
        in_specs=in_specs,
        out_specs=out_specs,
        out_shape=out_shape,
        interpret=interpret,
    )(a, hfull, w, b.reshape(1, fout), deg)


@functools.partial(jax.jit, static_argnames=("interpret",))
def kernel(node_feat, adjacency_matrix, node_degs, W0, b0, W1, b1, W2, b2,
           interpret=False):
    h1, a16 = _layer(adjacency_matrix, node_feat, node_degs, W0, b0,
                     block_m=400, out_dtype=jnp.bfloat16, emit_bf16_a=True,
                     interpret=interpret)
    h2 = _layer(a16, h1, node_degs, W1, b1, block_m=800,
                out_dtype=jnp.bfloat16, emit_bf16_a=False,
                interpret=interpret)
    h3 = _layer(a16, h2, node_degs, W2, b2, block_m=800,
                out_dtype=jnp.float32, emit_bf16_a=False,
                interpret=interpret)
    return h3
